# baseline (speedup 1.0000x reference)
import functools

import jax
import jax.numpy as jnp
from jax import lax
from jax.experimental import pallas as pl
from jax.experimental.pallas import tpu as pltpu
from jax.experimental.pallas import tpu_sc as plsc

_N = 10000
_E = 640000
_D = 128
_NP = 10240
_HN = 5120
_AR = 5128
_B = 80
_EPT = _E // 16
_NB = _EPT // _B
_PKR = 320
_PKH = 1288
_ZR = 80

_mesh = plsc.VectorSubcoreMesh(core_axis_name="c", subcore_axis_name="s",
                               num_cores=2, num_subcores=16)


def _zero2d(ref, rows):
    def body(t, _):
        ref[t // 8, pl.ds(lax.rem(t, 8) * 16, 16)] = jnp.zeros((16,),
                                                               jnp.float32)
        return 0
    lax.fori_loop(0, rows * 8, body, 0)


def _unpack(pkv, j, s80, d80, soff, lo):
    for k in range(_B // 16):
        flat = j * _B + k * 16
        v = pkv[flat >> 7, pl.ds(flat & 127, 16)]
        if s80 is not None:
            s80[pl.ds(k * 16, 16)] = (v & 0xFFFF) + soff
        d = (v >> 16) - lo
        d = jnp.where((d >= 0) & (d < _HN), d, _HN)
        d80[pl.ds(k * 16, 16)] = d


def _acc_zero(acc_sh, zb, sid):
    for k in range(320 // _ZR):
        pltpu.sync_copy(zb.at[pl.ds(0, _ZR)],
                        acc_sh.at[pl.ds(sid * 320 + k * _ZR, _ZR)])


def _acc_writeout(acc_sh, zb, out_ref, sid, row0):
    for k in range(320 // _ZR):
        pltpu.sync_copy(acc_sh.at[pl.ds(sid * 320 + k * _ZR, _ZR)],
                        zb.at[pl.ds(0, _ZR)])
        pltpu.sync_copy(zb.at[pl.ds(0, _ZR)],
                        out_ref.at[pl.ds(row0 + sid * 320 + k * _ZR, _ZR)])


@functools.partial(
    pl.kernel,
    out_type=jax.ShapeDtypeStruct((2, _NP, _D), jnp.float32),
    mesh=_mesh,
    scratch_types=[
        pltpu.VMEM((_PKR, 128), jnp.int32),
        pltpu.VMEM((_B,), jnp.int32),
        pltpu.VMEM((_B,), jnp.int32),
        pltpu.VMEM((_B, _D), jnp.float32),
        pltpu.VMEM((128, _D), jnp.float32),
        pltpu.VMEM_SHARED((_AR, _D), jnp.float32),
        pltpu.SemaphoreType.DMA,
        pltpu.SemaphoreType.DMA,
    ],
)
def _deg(pk_hbm, out_hbm, pkv, dda, ddb, ones_r, zb, acc_sh, ssa, ssb):
    cid = lax.axis_index("c")
    sid = lax.axis_index("s")
    lo = cid * _HN

    _zero2d(zb, 128)

    def obody(t, _):
        ones_r[t // 8, pl.ds(lax.rem(t, 8) * 16, 16)] = jnp.ones(
            (16,), jnp.float32)
        return 0
    lax.fori_loop(0, _B * 8, obody, 0)

    _acc_zero(acc_sh, zb, sid)
    pltpu.sync_copy(pk_hbm.at[sid, pl.ds(0, _PKR)], pkv)
    plsc.subcore_barrier()

    def unpack(j, d_idx):
        for k in range(_B // 16):
            flat = j * _B + k * 16
            v = pkv[flat >> 7, pl.ds(flat & 127, 16)]
            d = (v >> 16) - lo
            d = jnp.where((d >= 0) & (d < _HN), d, _HN)
            d_idx[pl.ds(k * 16, 16)] = d

    unpack(0, dda)
    pltpu.async_copy(ones_r, acc_sh.at[dda], ssa, add=True)

    def step(j, d_idx, ssem, o_d, o_ssem):
        @pl.when(j + 1 < _NB)
        def _():
            @pl.when(j >= 1)
            def _():
                pltpu.make_async_copy(ones_r, acc_sh.at[o_d],
                                      o_ssem).wait()
            unpack(j + 1, o_d)
            pltpu.async_copy(ones_r, acc_sh.at[o_d], o_ssem, add=True)

    def body(j, _):
        @pl.when(lax.rem(j, 2) == 0)
        def _():
            step(j, dda, ssa, ddb, ssb)

        @pl.when(lax.rem(j, 2) == 1)
        def _():
            step(j, ddb, ssb, dda, ssa)
        return 0
    lax.fori_loop(0, _NB, body, 0)
    pltpu.make_async_copy(ones_r, acc_sh.at[dda], ssa).wait()
    pltpu.make_async_copy(ones_r, acc_sh.at[ddb], ssb).wait()
    plsc.subcore_barrier()

    _acc_writeout(acc_sh, zb, out_hbm.at[0], sid, lo)


@functools.partial(
    pl.kernel,
    out_type=jax.ShapeDtypeStruct((2, _NP, _D), jnp.float32),
    mesh=_mesh,
    scratch_types=[
        pltpu.VMEM((_PKR, 128), jnp.int32),
        pltpu.VMEM((_B,), jnp.int32),
        pltpu.VMEM((_B,), jnp.int32),
        pltpu.VMEM((_B,), jnp.int32),
        pltpu.VMEM((_B,), jnp.int32),
        pltpu.VMEM((_B, _D), jnp.float32),
        pltpu.VMEM((_B, _D), jnp.float32),
        pltpu.VMEM((128, _D), jnp.float32),
        pltpu.VMEM_SHARED((_AR, _D), jnp.float32),
        pltpu.SemaphoreType.DMA,
        pltpu.SemaphoreType.DMA,
    ],
)
def _sn(pk_hbm, tab_hbm, out_hbm, pkv, sa, da, sb, db, ra, rb, zb,
        acc_sh, gsa, gsb):
    cid = lax.axis_index("c")
    sid = lax.axis_index("s")
    lo = cid * _HN

    _zero2d(zb, 128)
    _acc_zero(acc_sh, zb, sid)
    pltpu.sync_copy(pk_hbm.at[sid, pl.ds(0, _PKR)], pkv)
    plsc.subcore_barrier()

    def unpack(j, s_idx, d_idx):
        for k in range(_B // 16):
            flat = j * _B + k * 16
            v = pkv[flat >> 7, pl.ds(flat & 127, 16)]
            s_idx[pl.ds(k * 16, 16)] = v & 0xFFFF
            d = (v >> 16) - lo
            d = jnp.where((d >= 0) & (d < _HN), d, _HN)
            d_idx[pl.ds(k * 16, 16)] = d

    unpack(0, sa, da)
    pltpu.async_copy(tab_hbm.at[sa], ra, gsa)

    def step(j, s_idx, d_idx, rows, gsem, o_s, o_d, o_rows, o_gsem):
        @pl.when(j + 1 < _NB)
        def _():
            unpack(j + 1, o_s, o_d)
            pltpu.async_copy(tab_hbm.at[o_s], o_rows, o_gsem)
        pltpu.make_async_copy(tab_hbm.at[s_idx], rows, gsem).wait()
        pltpu.sync_copy(rows, acc_sh.at[d_idx], add=True)

    def body(j, _):
        @pl.when(lax.rem(j, 2) == 0)
        def _():
            step(j, sa, da, ra, gsa, sb, db, rb, gsb)

        @pl.when(lax.rem(j, 2) == 1)
        def _():
            step(j, sb, db, rb, gsb, sa, da, ra, gsa)
        return 0
    lax.fori_loop(0, _NB, body, 0)
    plsc.subcore_barrier()

    _acc_writeout(acc_sh, zb, out_hbm.at[0], sid, lo)


def _make_seg(lo):
    @functools.partial(
        pl.kernel,
        out_type=jax.ShapeDtypeStruct((2, _NP, _D), jnp.float32),
        mesh=_mesh,
        scratch_types=[
            pltpu.VMEM((_PKR, 128), jnp.int32),
            pltpu.VMEM((_B,), jnp.int32),
            pltpu.VMEM((_B,), jnp.int32),
            pltpu.VMEM((_B,), jnp.int32),
            pltpu.VMEM((_B,), jnp.int32),
            pltpu.VMEM((_B, _D), jnp.float32),
            pltpu.VMEM((_B, _D), jnp.float32),
            pltpu.VMEM((128, _D), jnp.float32),
            pltpu.VMEM_SHARED((_AR, _D), jnp.float32),
            pltpu.SemaphoreType.DMA,
            pltpu.SemaphoreType.DMA,
            pltpu.SemaphoreType.DMA,
            pltpu.SemaphoreType.DMA,
        ],
    )
    def seg(pk_hbm, tab_hbm, out_hbm, pkv, sa, da, sb, db, ra, rb, zb,
            acc_sh, gsa, gsb, ssa, ssb):
        cid = lax.axis_index("c")
        sid = lax.axis_index("s")
        soff = cid * _N

        _zero2d(zb, 128)
        _acc_zero(acc_sh, zb, sid)
        pltpu.sync_copy(pk_hbm.at[sid, pl.ds(0, _PKR)], pkv)
        plsc.subcore_barrier()

        def unpack(j, s_idx, d_idx):
            for k in range(_B // 16):
                flat = j * _B + k * 16
                v = pkv[flat >> 7, pl.ds(flat & 127, 16)]
                s_idx[pl.ds(k * 16, 16)] = (v & 0xFFFF) + soff
                d = (v >> 16) - lo
                d = jnp.where((d >= 0) & (d < _HN), d, _HN)
                d_idx[pl.ds(k * 16, 16)] = d

        unpack(0, sa, da)
        pltpu.async_copy(tab_hbm.at[sa], ra, gsa)

        def step(j, s_idx, d_idx, rows, gsem, ssem,
                 o_s, o_d, o_rows, o_gsem, o_ssem):
            pltpu.make_async_copy(tab_hbm.at[s_idx], rows, gsem).wait()
            pltpu.async_copy(rows, acc_sh.at[d_idx], ssem, add=True)

            @pl.when(j + 1 < _NB)
            def _():
                @pl.when(j >= 1)
                def _():
                    pltpu.make_async_copy(o_rows, acc_sh.at[o_d],
                                          o_ssem).wait()
                unpack(j + 1, o_s, o_d)
                pltpu.async_copy(tab_hbm.at[o_s], o_rows, o_gsem)

        def body(j, _):
            @pl.when(lax.rem(j, 2) == 0)
            def _():
                step(j, sa, da, ra, gsa, ssa, sb, db, rb, gsb, ssb)

            @pl.when(lax.rem(j, 2) == 1)
            def _():
                step(j, sb, db, rb, gsb, ssb, sa, da, ra, gsa, ssa)
            return 0
        lax.fori_loop(0, _NB, body, 0)
        pltpu.make_async_copy(ra, acc_sh.at[da], ssa).wait()
        pltpu.make_async_copy(rb, acc_sh.at[db], ssb).wait()
        plsc.subcore_barrier()

        _acc_writeout(acc_sh, zb, out_hbm.at[cid], sid, lo)

    return seg


_seg_lo = _make_seg(0)
_seg_hi = _make_seg(_HN)


_BLK = 400
_G = _N // _BLK


def _row_spec(w=_D):
    return pl.BlockSpec((_BLK, w), lambda i: (i, 0))


def _full_spec(r, c):
    return pl.BlockSpec((r, c), lambda i: (0, 0))


def _arecip_body(deg_ref, a_ref, inv_ref):
    d = jnp.maximum(deg_ref[...], 1.0)
    a_ref[...] = lax.rsqrt(d)
    inv_ref[...] = 1.0 / d


def _arecip(deg128):
    spec = pl.BlockSpec((_NP // 8, _D), lambda i: (i, 0))
    return pl.pallas_call(
        _arecip_body,
        grid=(8,),
        in_specs=[spec],
        out_specs=[spec, spec],
        out_shape=[jax.ShapeDtypeStruct((_NP, _D), jnp.float32),
                   jax.ShapeDtypeStruct((_NP, _D), jnp.float32)],
    )(deg128)


def _tca_body(x_ref, w0_ref, b0_ref, a_ref, h_ref, ah_ref):
    h = jnp.dot(x_ref[...], w0_ref[...],
                preferred_element_type=jnp.float32) + b0_ref[...]
    h_ref[...] = h
    ah_ref[...] = a_ref[...] * h


def _tca(x, W0, b0, a_col):
    return pl.pallas_call(
        _tca_body,
        grid=(_G,),
        in_specs=[_row_spec(), _full_spec(_D, _D), _full_spec(1, _D),
                  _row_spec(1)],
        out_specs=[_row_spec(), _row_spec()],
        out_shape=[jax.ShapeDtypeStruct((_N, _D), jnp.float32),
                   jax.ShapeDtypeStruct((_N, _D), jnp.float32)],
    )(x, W0, b0, a_col)


def _elu(v):
    return jnp.where(v > 0, v, jnp.exp(jnp.minimum(v, 0.0)) - 1.0)


def _tclayer_body(h_ref, s0_ref, s1_ref, a_ref, inv_ref, sn_ref,
                  wg_ref, bg_ref, wss_ref, wsn_ref, bs_ref,
                  wgin_ref, bgin_ref, w_ref, ho_ref, aho_ref):
    h = h_ref[...]
    s0 = s0_ref[...]
    a = a_ref[...]
    f32 = jnp.float32
    gcn = (jnp.dot(a * s1_ref[...], wg_ref[...], preferred_element_type=f32)
           + (a * sn_ref[...]) * bg_ref[...])
    sage = (jnp.dot(h, wss_ref[...], preferred_element_type=f32)
            + jnp.dot(inv_ref[...] * s0, wsn_ref[...],
                      preferred_element_type=f32) + bs_ref[...])
    gin = (jnp.dot(h + s0, wgin_ref[...], preferred_element_type=f32)
           + bgin_ref[...])
    xo = (w_ref[0] * _elu(gcn) + w_ref[1] * _elu(sage) + w_ref[2] * _elu(gin))
    ho_ref[...] = xo
    aho_ref[...] = a * xo


def _tclayer(h, S0, S1, a_col, inv_col, sn_col, Wg, bg, Wss, Wsn, bs,
             Wgin, bgin, naw):
    return pl.pallas_call(
        _tclayer_body,
        grid=(_G,),
        in_specs=[_row_spec(), _row_spec(), _row_spec(), _row_spec(1),
                  _row_spec(1), _row_spec(1),
                  _full_spec(_D, _D), _full_spec(1, _D),
                  _full_spec(_D, _D), _full_spec(_D, _D), _full_spec(1, _D),
                  _full_spec(_D, _D), _full_spec(1, _D),
                  pl.BlockSpec(memory_space=pltpu.MemorySpace.SMEM)],
        out_specs=[_row_spec(), _row_spec()],
        out_shape=[jax.ShapeDtypeStruct((_N, _D), jnp.float32),
                   jax.ShapeDtypeStruct((_N, _D), jnp.float32)],
    )(h, S0, S1, a_col, inv_col, sn_col, Wg, bg, Wss, Wsn, bs, Wgin, bgin,
      naw)


def _tcfinal_body(x1_ref, x2_ref, x3_ref, w1_ref, w2_ref, w3_ref, bla_ref,
                  wc_ref, bc_ref, k_ref, out_ref):
    f32 = jnp.float32
    x3 = x3_ref[...]
    sc1 = k_ref[0] * x1_ref[...]
    sc2 = k_ref[1] * x2_ref[...]
    op_max = jnp.maximum(jnp.maximum(x3, sc1), sc2)
    op_cat = (jnp.dot(x3, w1_ref[...], preferred_element_type=f32)
              + jnp.dot(sc1, w2_ref[...], preferred_element_type=f32)
              + jnp.dot(sc2, w3_ref[...], preferred_element_type=f32)
              + bla_ref[...])
    op_mean = (x3 + sc1 + sc2) / 3.0
    x5 = (k_ref[2] * jnp.maximum(op_max, 0.0)
          + k_ref[3] * jnp.maximum(op_cat, 0.0)
          + k_ref[4] * jnp.maximum(op_mean, 0.0))
    out_ref[...] = jnp.dot(x5, wc_ref[...],
                           preferred_element_type=f32) + bc_ref[...]


def _tcfinal(x1, x2, x3, W1, W2, W3, bla, Wc, bc, ks):
    c = Wc.shape[1]
    return pl.pallas_call(
        _tcfinal_body,
        grid=(_G,),
        in_specs=[_row_spec(), _row_spec(), _row_spec(),
                  _full_spec(_D, _D), _full_spec(_D, _D), _full_spec(_D, _D),
                  _full_spec(1, _D), _full_spec(_D, c), _full_spec(1, c),
                  pl.BlockSpec(memory_space=pltpu.MemorySpace.SMEM)],
        out_specs=[pl.BlockSpec((_BLK, c), lambda i: (i, 0))],
        out_shape=[jax.ShapeDtypeStruct((_N, c), jnp.float32)],
    )(x1, x2, x3, W1, W2, W3, bla, Wc, bc, ks)


def kernel(x, edge_index, W0, b0, Wg, bg, Wss, Wsn, bs, Wgin, bgin,
           Wla, bla, Wc, bc, na_alphas, sc_alphas, la_alphas):
    na_w = jax.nn.softmax(na_alphas, axis=-1)
    sc_w = jax.nn.softmax(sc_alphas, axis=-1)
    la_w = jax.nn.softmax(la_alphas, axis=-1)

    src = edge_index[0].astype(jnp.int32)
    dst = edge_index[1].astype(jnp.int32)
    pk = (src + dst * 65536).reshape(16, _EPT)
    pk = jnp.pad(pk, ((0, 0), (0, _PKH * 128 - _EPT)))
    pk = pk.reshape(16, _PKH, 128)

    deg128 = _deg(pk)[0]
    a128, inv128 = _arecip(deg128)
    a_col = a128[:_N, 0:1]
    inv_col = inv128[:_N, 0:1]

    h, ah = _tca(x, W0, b0.reshape(1, _D), a_col)
    sn_tab = jnp.concatenate([a128, a128], axis=0)
    sn_col = _sn(pk, sn_tab)[0][:_N, 0:1]

    xs = []
    for i in range(3):
        tab = jnp.concatenate([h, ah], axis=0)
        Sa = _seg_lo(pk, tab)
        Sb = _seg_hi(pk, tab)
        S0 = jnp.concatenate([Sa[0, :_HN], Sb[0, _HN:_N]], axis=0)
        S1 = jnp.concatenate([Sa[1, :_HN], Sb[1, _HN:_N]], axis=0)
        h, ah = _tclayer(h, S0, S1, a_col, inv_col, sn_col,
                         Wg[i], bg[i].reshape(1, _D), Wss[i], Wsn[i],
                         bs[i].reshape(1, _D), Wgin[i],
                         bgin[i].reshape(1, _D), na_w[i])
        xs.append(h)

    x1, x2, x3 = xs
    ks = jnp.stack([sc_w[0, 1], sc_w[1, 1],
                    la_w[0, 0], la_w[0, 1], la_w[0, 2]])
    (logits,) = _tcfinal(x1, x2, x3, Wla[0:_D], Wla[_D:2 * _D],
                         Wla[2 * _D:3 * _D], bla.reshape(1, _D), Wc,
                         bc.reshape(1, Wc.shape[1]), ks)
    return logits


# speedup vs baseline: 1.1372x; 1.1372x over previous
import functools

import jax
import jax.numpy as jnp
from jax import lax
from jax.experimental import pallas as pl
from jax.experimental.pallas import tpu as pltpu
from jax.experimental.pallas import tpu_sc as plsc

_N = 10000
_E = 640000
_D = 128
_NP = 10240
_HN = 5120
_AR = 5128
_B = 80
_EPT = _E // 16
_NB = _EPT // _B
_PKR = 320
_PKH = 1288
_ZR = 80

_mesh = plsc.VectorSubcoreMesh(core_axis_name="c", subcore_axis_name="s",
                               num_cores=2, num_subcores=16)


def _zero2d(ref, rows):
    def body(t, _):
        ref[t // 8, pl.ds(lax.rem(t, 8) * 16, 16)] = jnp.zeros((16,),
                                                               jnp.float32)
        return 0
    lax.fori_loop(0, rows * 8, body, 0)


def _unpack(pkv, j, s80, d80, soff, lo):
    for k in range(_B // 16):
        flat = j * _B + k * 16
        v = pkv[flat >> 7, pl.ds(flat & 127, 16)]
        if s80 is not None:
            s80[pl.ds(k * 16, 16)] = (v & 0xFFFF) + soff
        d = (v >> 16) - lo
        d = jnp.where((d >= 0) & (d < _HN), d, _HN)
        d80[pl.ds(k * 16, 16)] = d


def _acc_zero(acc_sh, zb, sid):
    for k in range(320 // _ZR):
        pltpu.sync_copy(zb.at[pl.ds(0, _ZR)],
                        acc_sh.at[pl.ds(sid * 320 + k * _ZR, _ZR)])


def _acc_writeout(acc_sh, zb, out_ref, sid, row0):
    for k in range(320 // _ZR):
        pltpu.sync_copy(acc_sh.at[pl.ds(sid * 320 + k * _ZR, _ZR)],
                        zb.at[pl.ds(0, _ZR)])
        pltpu.sync_copy(zb.at[pl.ds(0, _ZR)],
                        out_ref.at[pl.ds(row0 + sid * 320 + k * _ZR, _ZR)])


@functools.partial(
    pl.kernel,
    out_type=jax.ShapeDtypeStruct((2, _NP, _D), jnp.float32),
    mesh=_mesh,
    scratch_types=[
        pltpu.VMEM((_PKR, 128), jnp.int32),
        pltpu.VMEM((_B,), jnp.int32),
        pltpu.VMEM((_B,), jnp.int32),
        pltpu.VMEM((_B, _D), jnp.float32),
        pltpu.VMEM((128, _D), jnp.float32),
        pltpu.VMEM_SHARED((_AR, _D), jnp.float32),
        pltpu.SemaphoreType.DMA,
        pltpu.SemaphoreType.DMA,
    ],
)
def _deg(pk_hbm, out_hbm, pkv, dda, ddb, ones_r, zb, acc_sh, ssa, ssb):
    cid = lax.axis_index("c")
    sid = lax.axis_index("s")
    lo = cid * _HN

    _zero2d(zb, 128)

    def obody(t, _):
        ones_r[t // 8, pl.ds(lax.rem(t, 8) * 16, 16)] = jnp.ones(
            (16,), jnp.float32)
        return 0
    lax.fori_loop(0, _B * 8, obody, 0)

    _acc_zero(acc_sh, zb, sid)
    pltpu.sync_copy(pk_hbm.at[sid, pl.ds(0, _PKR)], pkv)
    plsc.subcore_barrier()

    def unpack(j, d_idx):
        for k in range(_B // 16):
            flat = j * _B + k * 16
            v = pkv[flat >> 7, pl.ds(flat & 127, 16)]
            d = (v >> 16) - lo
            d = jnp.where((d >= 0) & (d < _HN), d, _HN)
            d_idx[pl.ds(k * 16, 16)] = d

    unpack(0, dda)
    pltpu.async_copy(ones_r, acc_sh.at[dda], ssa, add=True)

    def step(j, d_idx, ssem, o_d, o_ssem):
        @pl.when(j + 1 < _NB)
        def _():
            @pl.when(j >= 1)
            def _():
                pltpu.make_async_copy(ones_r, acc_sh.at[o_d],
                                      o_ssem).wait()
            unpack(j + 1, o_d)
            pltpu.async_copy(ones_r, acc_sh.at[o_d], o_ssem, add=True)

    def body(j, _):
        @pl.when(lax.rem(j, 2) == 0)
        def _():
            step(j, dda, ssa, ddb, ssb)

        @pl.when(lax.rem(j, 2) == 1)
        def _():
            step(j, ddb, ssb, dda, ssa)
        return 0
    lax.fori_loop(0, _NB, body, 0)
    pltpu.make_async_copy(ones_r, acc_sh.at[dda], ssa).wait()
    pltpu.make_async_copy(ones_r, acc_sh.at[ddb], ssb).wait()
    plsc.subcore_barrier()

    _acc_writeout(acc_sh, zb, out_hbm.at[0], sid, lo)


@functools.partial(
    pl.kernel,
    out_type=jax.ShapeDtypeStruct((2, _NP, _D), jnp.float32),
    mesh=_mesh,
    scratch_types=[
        pltpu.VMEM((_PKR, 128), jnp.int32),
        pltpu.VMEM((_B,), jnp.int32),
        pltpu.VMEM((_B,), jnp.int32),
        pltpu.VMEM((_B,), jnp.int32),
        pltpu.VMEM((_B,), jnp.int32),
        pltpu.VMEM((_B, _D), jnp.float32),
        pltpu.VMEM((_B, _D), jnp.float32),
        pltpu.VMEM((128, _D), jnp.float32),
        pltpu.VMEM_SHARED((_AR, _D), jnp.float32),
        pltpu.SemaphoreType.DMA,
        pltpu.SemaphoreType.DMA,
    ],
)
def _sn(pk_hbm, tab_hbm, out_hbm, pkv, sa, da, sb, db, ra, rb, zb,
        acc_sh, gsa, gsb):
    cid = lax.axis_index("c")
    sid = lax.axis_index("s")
    lo = cid * _HN

    _zero2d(zb, 128)
    _acc_zero(acc_sh, zb, sid)
    pltpu.sync_copy(pk_hbm.at[sid, pl.ds(0, _PKR)], pkv)
    plsc.subcore_barrier()

    def unpack(j, s_idx, d_idx):
        for k in range(_B // 16):
            flat = j * _B + k * 16
            v = pkv[flat >> 7, pl.ds(flat & 127, 16)]
            s_idx[pl.ds(k * 16, 16)] = v & 0xFFFF
            d = (v >> 16) - lo
            d = jnp.where((d >= 0) & (d < _HN), d, _HN)
            d_idx[pl.ds(k * 16, 16)] = d

    unpack(0, sa, da)
    pltpu.async_copy(tab_hbm.at[sa], ra, gsa)

    def step(j, s_idx, d_idx, rows, gsem, o_s, o_d, o_rows, o_gsem):
        @pl.when(j + 1 < _NB)
        def _():
            unpack(j + 1, o_s, o_d)
            pltpu.async_copy(tab_hbm.at[o_s], o_rows, o_gsem)
        pltpu.make_async_copy(tab_hbm.at[s_idx], rows, gsem).wait()
        pltpu.sync_copy(rows, acc_sh.at[d_idx], add=True)

    def body(j, _):
        @pl.when(lax.rem(j, 2) == 0)
        def _():
            step(j, sa, da, ra, gsa, sb, db, rb, gsb)

        @pl.when(lax.rem(j, 2) == 1)
        def _():
            step(j, sb, db, rb, gsb, sa, da, ra, gsa)
        return 0
    lax.fori_loop(0, _NB, body, 0)
    plsc.subcore_barrier()

    _acc_writeout(acc_sh, zb, out_hbm.at[0], sid, lo)


def _make_seg(lo):
    @functools.partial(
        pl.kernel,
        out_type=jax.ShapeDtypeStruct((2, _NP, _D), jnp.float32),
        mesh=_mesh,
        scratch_types=[
            pltpu.VMEM((_PKR, 128), jnp.int32),
            pltpu.VMEM((_B,), jnp.int32),
            pltpu.VMEM((_B,), jnp.int32),
            pltpu.VMEM((_B,), jnp.int32),
            pltpu.VMEM((_B,), jnp.int32),
            pltpu.VMEM((_B, _D), jnp.float32),
            pltpu.VMEM((_B, _D), jnp.float32),
            pltpu.VMEM((128, _D), jnp.float32),
            pltpu.VMEM_SHARED((_AR, _D), jnp.float32),
            pltpu.SemaphoreType.DMA,
            pltpu.SemaphoreType.DMA,
        ],
    )
    def seg(pk_hbm, tab_hbm, out_hbm, pkv, sa, da, sb, db, ra, rb, zb,
            acc_sh, gsa, gsb):
        cid = lax.axis_index("c")
        sid = lax.axis_index("s")
        soff = cid * _N

        _zero2d(zb, 128)
        _acc_zero(acc_sh, zb, sid)
        pltpu.sync_copy(pk_hbm.at[sid, pl.ds(0, _PKR)], pkv)
        plsc.subcore_barrier()

        def unpack(j, s_idx, d_idx):
            for k in range(_B // 16):
                flat = j * _B + k * 16
                v = pkv[flat >> 7, pl.ds(flat & 127, 16)]
                s_idx[pl.ds(k * 16, 16)] = (v & 0xFFFF) + soff
                d = (v >> 16) - lo
                d = jnp.where((d >= 0) & (d < _HN), d, _HN)
                d_idx[pl.ds(k * 16, 16)] = d

        unpack(0, sa, da)
        pltpu.async_copy(tab_hbm.at[sa], ra, gsa)

        def step(j, s_idx, d_idx, rows, gsem, o_s, o_d, o_rows, o_gsem):
            @pl.when(j + 1 < _NB)
            def _():
                unpack(j + 1, o_s, o_d)
                pltpu.async_copy(tab_hbm.at[o_s], o_rows, o_gsem)
            pltpu.make_async_copy(tab_hbm.at[s_idx], rows, gsem).wait()
            pltpu.sync_copy(rows, acc_sh.at[d_idx], add=True)

        def body(j, _):
            @pl.when(lax.rem(j, 2) == 0)
            def _():
                step(j, sa, da, ra, gsa, sb, db, rb, gsb)

            @pl.when(lax.rem(j, 2) == 1)
            def _():
                step(j, sb, db, rb, gsb, sa, da, ra, gsa)
            return 0
        lax.fori_loop(0, _NB, body, 0)
        plsc.subcore_barrier()

        _acc_writeout(acc_sh, zb, out_hbm.at[cid], sid, lo)

    return seg


_seg_lo = _make_seg(0)
_seg_hi = _make_seg(_HN)


_BLK = 400
_G = _N // _BLK


def _row_spec(w=_D):
    return pl.BlockSpec((_BLK, w), lambda i: (i, 0))


def _full_spec(r, c):
    return pl.BlockSpec((r, c), lambda i: (0, 0))


def _arecip_body(deg_ref, a_ref, inv_ref):
    d = jnp.maximum(deg_ref[...], 1.0)
    a_ref[...] = lax.rsqrt(d)
    inv_ref[...] = 1.0 / d


def _arecip(deg128):
    spec = pl.BlockSpec((_NP // 8, _D), lambda i: (i, 0))
    return pl.pallas_call(
        _arecip_body,
        grid=(8,),
        in_specs=[spec],
        out_specs=[spec, spec],
        out_shape=[jax.ShapeDtypeStruct((_NP, _D), jnp.float32),
                   jax.ShapeDtypeStruct((_NP, _D), jnp.float32)],
    )(deg128)


def _tca_body(x_ref, w0_ref, b0_ref, a_ref, h_ref, ah_ref):
    h = jnp.dot(x_ref[...], w0_ref[...],
                preferred_element_type=jnp.float32) + b0_ref[...]
    h_ref[...] = h
    ah_ref[...] = a_ref[...] * h


def _tca(x, W0, b0, a_col):
    return pl.pallas_call(
        _tca_body,
        grid=(_G,),
        in_specs=[_row_spec(), _full_spec(_D, _D), _full_spec(1, _D),
                  _row_spec(1)],
        out_specs=[_row_spec(), _row_spec()],
        out_shape=[jax.ShapeDtypeStruct((_N, _D), jnp.float32),
                   jax.ShapeDtypeStruct((_N, _D), jnp.float32)],
    )(x, W0, b0, a_col)


def _elu(v):
    return jnp.where(v > 0, v, jnp.exp(jnp.minimum(v, 0.0)) - 1.0)


def _tclayer_body(h_ref, s0_ref, s1_ref, a_ref, inv_ref, sn_ref,
                  wg_ref, bg_ref, wss_ref, wsn_ref, bs_ref,
                  wgin_ref, bgin_ref, w_ref, ho_ref, aho_ref):
    h = h_ref[...]
    s0 = s0_ref[...]
    a = a_ref[...]
    f32 = jnp.float32
    gcn = (jnp.dot(a * s1_ref[...], wg_ref[...], preferred_element_type=f32)
           + (a * sn_ref[...]) * bg_ref[...])
    sage = (jnp.dot(h, wss_ref[...], preferred_element_type=f32)
            + jnp.dot(inv_ref[...] * s0, wsn_ref[...],
                      preferred_element_type=f32) + bs_ref[...])
    gin = (jnp.dot(h + s0, wgin_ref[...], preferred_element_type=f32)
           + bgin_ref[...])
    xo = (w_ref[0] * _elu(gcn) + w_ref[1] * _elu(sage) + w_ref[2] * _elu(gin))
    ho_ref[...] = xo
    aho_ref[...] = a * xo


def _tclayer(h, S0, S1, a_col, inv_col, sn_col, Wg, bg, Wss, Wsn, bs,
             Wgin, bgin, naw):
    return pl.pallas_call(
        _tclayer_body,
        grid=(_G,),
        in_specs=[_row_spec(), _row_spec(), _row_spec(), _row_spec(1),
                  _row_spec(1), _row_spec(1),
                  _full_spec(_D, _D), _full_spec(1, _D),
                  _full_spec(_D, _D), _full_spec(_D, _D), _full_spec(1, _D),
                  _full_spec(_D, _D), _full_spec(1, _D),
                  pl.BlockSpec(memory_space=pltpu.MemorySpace.SMEM)],
        out_specs=[_row_spec(), _row_spec()],
        out_shape=[jax.ShapeDtypeStruct((_N, _D), jnp.float32),
                   jax.ShapeDtypeStruct((_N, _D), jnp.float32)],
    )(h, S0, S1, a_col, inv_col, sn_col, Wg, bg, Wss, Wsn, bs, Wgin, bgin,
      naw)


def _tcfinal_body(x1_ref, x2_ref, x3_ref, w1_ref, w2_ref, w3_ref, bla_ref,
                  wc_ref, bc_ref, k_ref, out_ref):
    f32 = jnp.float32
    x3 = x3_ref[...]
    sc1 = k_ref[0] * x1_ref[...]
    sc2 = k_ref[1] * x2_ref[...]
    op_max = jnp.maximum(jnp.maximum(x3, sc1), sc2)
    op_cat = (jnp.dot(x3, w1_ref[...], preferred_element_type=f32)
              + jnp.dot(sc1, w2_ref[...], preferred_element_type=f32)
              + jnp.dot(sc2, w3_ref[...], preferred_element_type=f32)
              + bla_ref[...])
    op_mean = (x3 + sc1 + sc2) / 3.0
    x5 = (k_ref[2] * jnp.maximum(op_max, 0.0)
          + k_ref[3] * jnp.maximum(op_cat, 0.0)
          + k_ref[4] * jnp.maximum(op_mean, 0.0))
    out_ref[...] = jnp.dot(x5, wc_ref[...],
                           preferred_element_type=f32) + bc_ref[...]


def _tcfinal(x1, x2, x3, W1, W2, W3, bla, Wc, bc, ks):
    c = Wc.shape[1]
    return pl.pallas_call(
        _tcfinal_body,
        grid=(_G,),
        in_specs=[_row_spec(), _row_spec(), _row_spec(),
                  _full_spec(_D, _D), _full_spec(_D, _D), _full_spec(_D, _D),
                  _full_spec(1, _D), _full_spec(_D, c), _full_spec(1, c),
                  pl.BlockSpec(memory_space=pltpu.MemorySpace.SMEM)],
        out_specs=[pl.BlockSpec((_BLK, c), lambda i: (i, 0))],
        out_shape=[jax.ShapeDtypeStruct((_N, c), jnp.float32)],
    )(x1, x2, x3, W1, W2, W3, bla, Wc, bc, ks)


def kernel(x, edge_index, W0, b0, Wg, bg, Wss, Wsn, bs, Wgin, bgin,
           Wla, bla, Wc, bc, na_alphas, sc_alphas, la_alphas):
    na_w = jax.nn.softmax(na_alphas, axis=-1)
    sc_w = jax.nn.softmax(sc_alphas, axis=-1)
    la_w = jax.nn.softmax(la_alphas, axis=-1)

    src = edge_index[0].astype(jnp.int32)
    dst = edge_index[1].astype(jnp.int32)
    pk = (src + dst * 65536).reshape(16, _EPT)
    pk = jnp.pad(pk, ((0, 0), (0, _PKH * 128 - _EPT)))
    pk = pk.reshape(16, _PKH, 128)

    deg128 = _deg(pk)[0]
    a128, inv128 = _arecip(deg128)
    a_col = a128[:_N, 0:1]
    inv_col = inv128[:_N, 0:1]

    h, ah = _tca(x, W0, b0.reshape(1, _D), a_col)
    sn_tab = jnp.concatenate([a128, a128], axis=0)
    sn_col = _sn(pk, sn_tab)[0][:_N, 0:1]

    xs = []
    for i in range(3):
        tab = jnp.concatenate([h, ah], axis=0)
        Sa = _seg_lo(pk, tab)
        Sb = _seg_hi(pk, tab)
        S0 = jnp.concatenate([Sa[0, :_HN], Sb[0, _HN:_N]], axis=0)
        S1 = jnp.concatenate([Sa[1, :_HN], Sb[1, _HN:_N]], axis=0)
        h, ah = _tclayer(h, S0, S1, a_col, inv_col, sn_col,
                         Wg[i], bg[i].reshape(1, _D), Wss[i], Wsn[i],
                         bs[i].reshape(1, _D), Wgin[i],
                         bgin[i].reshape(1, _D), na_w[i])
        xs.append(h)

    x1, x2, x3 = xs
    ks = jnp.stack([sc_w[0, 1], sc_w[1, 1],
                    la_w[0, 0], la_w[0, 1], la_w[0, 2]])
    (logits,) = _tcfinal(x1, x2, x3, Wla[0:_D], Wla[_D:2 * _D],
                         Wla[2 * _D:3 * _D], bla.reshape(1, _D), Wc,
                         bc.reshape(1, Wc.shape[1]), ks)
    return logits


# speedup vs baseline: 1.1379x; 1.0006x over previous
import functools

import jax
import jax.numpy as jnp
from jax import lax
from jax.experimental import pallas as pl
from jax.experimental.pallas import tpu as pltpu
from jax.experimental.pallas import tpu_sc as plsc

_N = 10000
_E = 640000
_D = 128
_NP = 10240
_HN = 5120
_AR = 5128
_B = 80
_EPT = _E // 16
_NB = _EPT // _B
_PKR = 320
_PKH = 1288
_ZR = 80

_mesh = plsc.VectorSubcoreMesh(core_axis_name="c", subcore_axis_name="s",
                               num_cores=2, num_subcores=16)


def _zero2d(ref, rows):
    def body(t, _):
        ref[t // 8, pl.ds(lax.rem(t, 8) * 16, 16)] = jnp.zeros((16,),
                                                               jnp.float32)
        return 0
    lax.fori_loop(0, rows * 8, body, 0)


def _unpack(pkv, j, s80, d80, soff, lo):
    for k in range(_B // 16):
        flat = j * _B + k * 16
        v = pkv[flat >> 7, pl.ds(flat & 127, 16)]
        if s80 is not None:
            s80[pl.ds(k * 16, 16)] = (v & 0xFFFF) + soff
        d = (v >> 16) - lo
        d = jnp.where((d >= 0) & (d < _HN), d, _HN)
        d80[pl.ds(k * 16, 16)] = d


def _acc_zero(acc_sh, zb, sid):
    for k in range(320 // _ZR):
        pltpu.sync_copy(zb.at[pl.ds(0, _ZR)],
                        acc_sh.at[pl.ds(sid * 320 + k * _ZR, _ZR)])


def _acc_writeout(acc_sh, zb, out_ref, sid, row0):
    for k in range(320 // _ZR):
        pltpu.sync_copy(acc_sh.at[pl.ds(sid * 320 + k * _ZR, _ZR)],
                        zb.at[pl.ds(0, _ZR)])
        pltpu.sync_copy(zb.at[pl.ds(0, _ZR)],
                        out_ref.at[pl.ds(row0 + sid * 320 + k * _ZR, _ZR)])


@functools.partial(
    pl.kernel,
    out_type=jax.ShapeDtypeStruct((2, _NP, _D), jnp.float32),
    mesh=_mesh,
    scratch_types=[
        pltpu.VMEM((_PKR, 128), jnp.int32),
        pltpu.VMEM((4, _B), jnp.int32),
        pltpu.VMEM((_B, _D), jnp.float32),
        pltpu.VMEM((128, _D), jnp.float32),
        pltpu.VMEM_SHARED((_AR, _D), jnp.float32),
        pltpu.SemaphoreType.DMA,
    ],
)
def _deg(pk_hbm, out_hbm, pkv, dd, ones_r, zb, acc_sh, ssem):
    cid = lax.axis_index("c")
    sid = lax.axis_index("s")
    lo = cid * _HN

    _zero2d(zb, 128)

    def obody(t, _):
        ones_r[t // 8, pl.ds(lax.rem(t, 8) * 16, 16)] = jnp.ones(
            (16,), jnp.float32)
        return 0
    lax.fori_loop(0, _B * 8, obody, 0)

    _acc_zero(acc_sh, zb, sid)
    pltpu.sync_copy(pk_hbm.at[sid, pl.ds(0, _PKR)], pkv)
    plsc.subcore_barrier()

    def unpack(j, slot):
        for k in range(_B // 16):
            flat = j * _B + k * 16
            v = pkv[flat >> 7, pl.ds(flat & 127, 16)]
            d = (v >> 16) - lo
            d = jnp.where((d >= 0) & (d < _HN), d, _HN)
            dd[slot, pl.ds(k * 16, 16)] = d

    def prime(j, _):
        unpack(j, lax.rem(j, 4))
        pltpu.async_copy(ones_r, acc_sh.at[dd.at[lax.rem(j, 4)]], ssem,
                         add=True)
        return 0
    lax.fori_loop(0, 3, prime, 0)

    def body(j, _):
        slot = lax.rem(j, 4)
        @pl.when(j >= 3)
        def _():
            pltpu.make_async_copy(ones_r, acc_sh.at[dd.at[slot]],
                                  ssem).wait()
        unpack(j, slot)
        pltpu.async_copy(ones_r, acc_sh.at[dd.at[slot]], ssem, add=True)
        return 0
    lax.fori_loop(3, _NB, body, 0)

    def drain(j, _):
        pltpu.make_async_copy(ones_r, acc_sh.at[dd.at[0]], ssem).wait()
        return 0
    lax.fori_loop(0, 3, drain, 0)
    plsc.subcore_barrier()

    _acc_writeout(acc_sh, zb, out_hbm.at[0], sid, lo)


@functools.partial(
    pl.kernel,
    out_type=jax.ShapeDtypeStruct((2, _NP, _D), jnp.float32),
    mesh=_mesh,
    scratch_types=[
        pltpu.VMEM((_PKR, 128), jnp.int32),
        pltpu.VMEM((_B,), jnp.int32),
        pltpu.VMEM((_B,), jnp.int32),
        pltpu.VMEM((_B,), jnp.int32),
        pltpu.VMEM((_B,), jnp.int32),
        pltpu.VMEM((_B, _D), jnp.float32),
        pltpu.VMEM((_B, _D), jnp.float32),
        pltpu.VMEM((128, _D), jnp.float32),
        pltpu.VMEM_SHARED((_AR, _D), jnp.float32),
        pltpu.SemaphoreType.DMA,
        pltpu.SemaphoreType.DMA,
    ],
)
def _sn(pk_hbm, tab_hbm, out_hbm, pkv, sa, da, sb, db, ra, rb, zb,
        acc_sh, gsa, gsb):
    cid = lax.axis_index("c")
    sid = lax.axis_index("s")
    lo = cid * _HN

    _zero2d(zb, 128)
    _acc_zero(acc_sh, zb, sid)
    pltpu.sync_copy(pk_hbm.at[sid, pl.ds(0, _PKR)], pkv)
    plsc.subcore_barrier()

    def unpack(j, s_idx, d_idx):
        for k in range(_B // 16):
            flat = j * _B + k * 16
            v = pkv[flat >> 7, pl.ds(flat & 127, 16)]
            s_idx[pl.ds(k * 16, 16)] = v & 0xFFFF
            d = (v >> 16) - lo
            d = jnp.where((d >= 0) & (d < _HN), d, _HN)
            d_idx[pl.ds(k * 16, 16)] = d

    unpack(0, sa, da)
    pltpu.async_copy(tab_hbm.at[sa], ra, gsa)

    def step(j, s_idx, d_idx, rows, gsem, o_s, o_d, o_rows, o_gsem):
        @pl.when(j + 1 < _NB)
        def _():
            unpack(j + 1, o_s, o_d)
            pltpu.async_copy(tab_hbm.at[o_s], o_rows, o_gsem)
        pltpu.make_async_copy(tab_hbm.at[s_idx], rows, gsem).wait()
        pltpu.sync_copy(rows, acc_sh.at[d_idx], add=True)

    def body(j, _):
        @pl.when(lax.rem(j, 2) == 0)
        def _():
            step(j, sa, da, ra, gsa, sb, db, rb, gsb)

        @pl.when(lax.rem(j, 2) == 1)
        def _():
            step(j, sb, db, rb, gsb, sa, da, ra, gsa)
        return 0
    lax.fori_loop(0, _NB, body, 0)
    plsc.subcore_barrier()

    _acc_writeout(acc_sh, zb, out_hbm.at[0], sid, lo)


def _make_seg(lo):
    @functools.partial(
        pl.kernel,
        out_type=jax.ShapeDtypeStruct((2, _NP, _D), jnp.float32),
        mesh=_mesh,
        scratch_types=[
            pltpu.VMEM((_PKR, 128), jnp.int32),
            pltpu.VMEM((_B,), jnp.int32),
            pltpu.VMEM((_B,), jnp.int32),
            pltpu.VMEM((_B,), jnp.int32),
            pltpu.VMEM((_B,), jnp.int32),
            pltpu.VMEM((_B, _D), jnp.float32),
            pltpu.VMEM((_B, _D), jnp.float32),
            pltpu.VMEM((128, _D), jnp.float32),
            pltpu.VMEM_SHARED((_AR, _D), jnp.float32),
            pltpu.SemaphoreType.DMA,
            pltpu.SemaphoreType.DMA,
        ],
    )
    def seg(pk_hbm, tab_hbm, out_hbm, pkv, sa, da, sb, db, ra, rb, zb,
            acc_sh, gsa, gsb):
        cid = lax.axis_index("c")
        sid = lax.axis_index("s")
        soff = cid * _N

        _zero2d(zb, 128)
        _acc_zero(acc_sh, zb, sid)
        pltpu.sync_copy(pk_hbm.at[sid, pl.ds(0, _PKR)], pkv)
        plsc.subcore_barrier()

        def unpack(j, s_idx, d_idx):
            for k in range(_B // 16):
                flat = j * _B + k * 16
                v = pkv[flat >> 7, pl.ds(flat & 127, 16)]
                s_idx[pl.ds(k * 16, 16)] = (v & 0xFFFF) + soff
                d = (v >> 16) - lo
                d = jnp.where((d >= 0) & (d < _HN), d, _HN)
                d_idx[pl.ds(k * 16, 16)] = d

        unpack(0, sa, da)
        pltpu.async_copy(tab_hbm.at[sa], ra, gsa)

        def step(j, s_idx, d_idx, rows, gsem, o_s, o_d, o_rows, o_gsem):
            @pl.when(j + 1 < _NB)
            def _():
                unpack(j + 1, o_s, o_d)
                pltpu.async_copy(tab_hbm.at[o_s], o_rows, o_gsem)
            pltpu.make_async_copy(tab_hbm.at[s_idx], rows, gsem).wait()
            pltpu.sync_copy(rows, acc_sh.at[d_idx], add=True)

        def body(j, _):
            @pl.when(lax.rem(j, 2) == 0)
            def _():
                step(j, sa, da, ra, gsa, sb, db, rb, gsb)

            @pl.when(lax.rem(j, 2) == 1)
            def _():
                step(j, sb, db, rb, gsb, sa, da, ra, gsa)
            return 0
        lax.fori_loop(0, _NB, body, 0)
        plsc.subcore_barrier()

        _acc_writeout(acc_sh, zb, out_hbm.at[cid], sid, lo)

    return seg


_seg_lo = _make_seg(0)
_seg_hi = _make_seg(_HN)


_BLK = 400
_G = _N // _BLK


def _row_spec(w=_D):
    return pl.BlockSpec((_BLK, w), lambda i: (i, 0))


def _full_spec(r, c):
    return pl.BlockSpec((r, c), lambda i: (0, 0))


def _arecip_body(deg_ref, a_ref, inv_ref):
    d = jnp.maximum(deg_ref[...], 1.0)
    a_ref[...] = lax.rsqrt(d)
    inv_ref[...] = 1.0 / d


def _arecip(deg128):
    spec = pl.BlockSpec((_NP // 8, _D), lambda i: (i, 0))
    return pl.pallas_call(
        _arecip_body,
        grid=(8,),
        in_specs=[spec],
        out_specs=[spec, spec],
        out_shape=[jax.ShapeDtypeStruct((_NP, _D), jnp.float32),
                   jax.ShapeDtypeStruct((_NP, _D), jnp.float32)],
    )(deg128)


def _tca_body(x_ref, w0_ref, b0_ref, a_ref, h_ref, ah_ref):
    h = jnp.dot(x_ref[...], w0_ref[...],
                preferred_element_type=jnp.float32) + b0_ref[...]
    h_ref[...] = h
    ah_ref[...] = a_ref[...] * h


def _tca(x, W0, b0, a_col):
    return pl.pallas_call(
        _tca_body,
        grid=(_G,),
        in_specs=[_row_spec(), _full_spec(_D, _D), _full_spec(1, _D),
                  _row_spec(1)],
        out_specs=[_row_spec(), _row_spec()],
        out_shape=[jax.ShapeDtypeStruct((_N, _D), jnp.float32),
                   jax.ShapeDtypeStruct((_N, _D), jnp.float32)],
    )(x, W0, b0, a_col)


def _elu(v):
    return jnp.where(v > 0, v, jnp.exp(jnp.minimum(v, 0.0)) - 1.0)


def _tclayer_body(h_ref, s0_ref, s1_ref, a_ref, inv_ref, sn_ref,
                  wg_ref, bg_ref, wss_ref, wsn_ref, bs_ref,
                  wgin_ref, bgin_ref, w_ref, ho_ref, aho_ref):
    h = h_ref[...]
    s0 = s0_ref[...]
    a = a_ref[...]
    f32 = jnp.float32
    gcn = (jnp.dot(a * s1_ref[...], wg_ref[...], preferred_element_type=f32)
           + (a * sn_ref[...]) * bg_ref[...])
    sage = (jnp.dot(h, wss_ref[...], preferred_element_type=f32)
            + jnp.dot(inv_ref[...] * s0, wsn_ref[...],
                      preferred_element_type=f32) + bs_ref[...])
    gin = (jnp.dot(h + s0, wgin_ref[...], preferred_element_type=f32)
           + bgin_ref[...])
    xo = (w_ref[0] * _elu(gcn) + w_ref[1] * _elu(sage) + w_ref[2] * _elu(gin))
    ho_ref[...] = xo
    aho_ref[...] = a * xo


def _tclayer(h, S0, S1, a_col, inv_col, sn_col, Wg, bg, Wss, Wsn, bs,
             Wgin, bgin, naw):
    return pl.pallas_call(
        _tclayer_body,
        grid=(_G,),
        in_specs=[_row_spec(), _row_spec(), _row_spec(), _row_spec(1),
                  _row_spec(1), _row_spec(1),
                  _full_spec(_D, _D), _full_spec(1, _D),
                  _full_spec(_D, _D), _full_spec(_D, _D), _full_spec(1, _D),
                  _full_spec(_D, _D), _full_spec(1, _D),
                  pl.BlockSpec(memory_space=pltpu.MemorySpace.SMEM)],
        out_specs=[_row_spec(), _row_spec()],
        out_shape=[jax.ShapeDtypeStruct((_N, _D), jnp.float32),
                   jax.ShapeDtypeStruct((_N, _D), jnp.float32)],
    )(h, S0, S1, a_col, inv_col, sn_col, Wg, bg, Wss, Wsn, bs, Wgin, bgin,
      naw)


def _tcfinal_body(x1_ref, x2_ref, x3_ref, w1_ref, w2_ref, w3_ref, bla_ref,
                  wc_ref, bc_ref, k_ref, out_ref):
    f32 = jnp.float32
    x3 = x3_ref[...]
    sc1 = k_ref[0] * x1_ref[...]
    sc2 = k_ref[1] * x2_ref[...]
    op_max = jnp.maximum(jnp.maximum(x3, sc1), sc2)
    op_cat = (jnp.dot(x3, w1_ref[...], preferred_element_type=f32)
              + jnp.dot(sc1, w2_ref[...], preferred_element_type=f32)
              + jnp.dot(sc2, w3_ref[...], preferred_element_type=f32)
              + bla_ref[...])
    op_mean = (x3 + sc1 + sc2) / 3.0
    x5 = (k_ref[2] * jnp.maximum(op_max, 0.0)
          + k_ref[3] * jnp.maximum(op_cat, 0.0)
          + k_ref[4] * jnp.maximum(op_mean, 0.0))
    out_ref[...] = jnp.dot(x5, wc_ref[...],
                           preferred_element_type=f32) + bc_ref[...]


def _tcfinal(x1, x2, x3, W1, W2, W3, bla, Wc, bc, ks):
    c = Wc.shape[1]
    return pl.pallas_call(
        _tcfinal_body,
        grid=(_G,),
        in_specs=[_row_spec(), _row_spec(), _row_spec(),
                  _full_spec(_D, _D), _full_spec(_D, _D), _full_spec(_D, _D),
                  _full_spec(1, _D), _full_spec(_D, c), _full_spec(1, c),
                  pl.BlockSpec(memory_space=pltpu.MemorySpace.SMEM)],
        out_specs=[pl.BlockSpec((_BLK, c), lambda i: (i, 0))],
        out_shape=[jax.ShapeDtypeStruct((_N, c), jnp.float32)],
    )(x1, x2, x3, W1, W2, W3, bla, Wc, bc, ks)


def kernel(x, edge_index, W0, b0, Wg, bg, Wss, Wsn, bs, Wgin, bgin,
           Wla, bla, Wc, bc, na_alphas, sc_alphas, la_alphas):
    na_w = jax.nn.softmax(na_alphas, axis=-1)
    sc_w = jax.nn.softmax(sc_alphas, axis=-1)
    la_w = jax.nn.softmax(la_alphas, axis=-1)

    src = edge_index[0].astype(jnp.int32)
    dst = edge_index[1].astype(jnp.int32)
    pk = (src + dst * 65536).reshape(16, _EPT)
    pk = jnp.pad(pk, ((0, 0), (0, _PKH * 128 - _EPT)))
    pk = pk.reshape(16, _PKH, 128)

    deg128 = _deg(pk)[0]
    a128, inv128 = _arecip(deg128)
    a_col = a128[:_N, 0:1]
    inv_col = inv128[:_N, 0:1]

    h, ah = _tca(x, W0, b0.reshape(1, _D), a_col)
    sn_tab = jnp.concatenate([a128, a128], axis=0)
    sn_col = _sn(pk, sn_tab)[0][:_N, 0:1]

    xs = []
    for i in range(3):
        tab = jnp.concatenate([h, ah], axis=0)
        Sa = _seg_lo(pk, tab)
        Sb = _seg_hi(pk, tab)
        S0 = jnp.concatenate([Sa[0, :_HN], Sb[0, _HN:_N]], axis=0)
        S1 = jnp.concatenate([Sa[1, :_HN], Sb[1, _HN:_N]], axis=0)
        h, ah = _tclayer(h, S0, S1, a_col, inv_col, sn_col,
                         Wg[i], bg[i].reshape(1, _D), Wss[i], Wsn[i],
                         bs[i].reshape(1, _D), Wgin[i],
                         bgin[i].reshape(1, _D), na_w[i])
        xs.append(h)

    x1, x2, x3 = xs
    ks = jnp.stack([sc_w[0, 1], sc_w[1, 1],
                    la_w[0, 0], la_w[0, 1], la_w[0, 2]])
    (logits,) = _tcfinal(x1, x2, x3, Wla[0:_D], Wla[_D:2 * _D],
                         Wla[2 * _D:3 * _D], bla.reshape(1, _D), Wc,
                         bc.reshape(1, Wc.shape[1]), ks)
    return logits


# speedup vs baseline: 1.1458x; 1.0070x over previous
import functools

import jax
import jax.numpy as jnp
from jax import lax
from jax.experimental import pallas as pl
from jax.experimental.pallas import tpu as pltpu
from jax.experimental.pallas import tpu_sc as plsc

_N = 10000
_E = 640000
_D = 128
_NP = 10240
_HN = 5120
_AR = 5128
_B = 80
_EPT = _E // 16
_NB = _EPT // _B
_PKR = 320
_PKH = 1288
_ZR = 80

_mesh = plsc.VectorSubcoreMesh(core_axis_name="c", subcore_axis_name="s",
                               num_cores=2, num_subcores=16)


def _zero2d(ref, rows, w=8):
    def body(t, _):
        ref[t // w, pl.ds(lax.rem(t, w) * 16, 16)] = jnp.zeros((16,),
                                                               jnp.float32)
        return 0
    lax.fori_loop(0, rows * w, body, 0)


def _unpack(pkv, j, s80, d80, soff, lo):
    for k in range(_B // 16):
        flat = j * _B + k * 16
        v = pkv[flat >> 7, pl.ds(flat & 127, 16)]
        if s80 is not None:
            s80[pl.ds(k * 16, 16)] = (v & 0xFFFF) + soff
        d = (v >> 16) - lo
        d = jnp.where((d >= 0) & (d < _HN), d, _HN)
        d80[pl.ds(k * 16, 16)] = d


def _acc_zero(acc_sh, zb, sid):
    for k in range(320 // _ZR):
        pltpu.sync_copy(zb.at[pl.ds(0, _ZR)],
                        acc_sh.at[pl.ds(sid * 320 + k * _ZR, _ZR)])


def _acc_writeout(acc_sh, zb, out_ref, sid, row0):
    for k in range(320 // _ZR):
        pltpu.sync_copy(acc_sh.at[pl.ds(sid * 320 + k * _ZR, _ZR)],
                        zb.at[pl.ds(0, _ZR)])
        pltpu.sync_copy(zb.at[pl.ds(0, _ZR)],
                        out_ref.at[pl.ds(row0 + sid * 320 + k * _ZR, _ZR)])


@functools.partial(
    pl.kernel,
    out_type=jax.ShapeDtypeStruct((2, _NP, 16), jnp.float32),
    mesh=_mesh,
    scratch_types=[
        pltpu.VMEM((_PKR, 128), jnp.int32),
        pltpu.VMEM((4, _B), jnp.int32),
        pltpu.VMEM((_B, 16), jnp.float32),
        pltpu.VMEM((128, 16), jnp.float32),
        pltpu.VMEM_SHARED((_AR, 16), jnp.float32),
        pltpu.SemaphoreType.DMA,
    ],
)
def _deg(pk_hbm, out_hbm, pkv, dd, ones_r, zb, acc_sh, ssem):
    cid = lax.axis_index("c")
    sid = lax.axis_index("s")
    lo = cid * _HN

    _zero2d(zb, 128, w=1)

    def obody(t, _):
        ones_r[t, pl.ds(0, 16)] = jnp.ones((16,), jnp.float32)
        return 0
    lax.fori_loop(0, _B, obody, 0)

    _acc_zero(acc_sh, zb, sid)
    pltpu.sync_copy(pk_hbm.at[sid, pl.ds(0, _PKR)], pkv)
    plsc.subcore_barrier()

    def unpack(j, slot):
        for k in range(_B // 16):
            flat = j * _B + k * 16
            v = pkv[flat >> 7, pl.ds(flat & 127, 16)]
            d = (v >> 16) - lo
            d = jnp.where((d >= 0) & (d < _HN), d, _HN)
            dd[slot, pl.ds(k * 16, 16)] = d

    def prime(j, _):
        unpack(j, lax.rem(j, 4))
        pltpu.async_copy(ones_r, acc_sh.at[dd.at[lax.rem(j, 4)]], ssem,
                         add=True)
        return 0
    lax.fori_loop(0, 3, prime, 0)

    def body(j, _):
        slot = lax.rem(j, 4)
        @pl.when(j >= 3)
        def _():
            pltpu.make_async_copy(ones_r, acc_sh.at[dd.at[slot]],
                                  ssem).wait()
        unpack(j, slot)
        pltpu.async_copy(ones_r, acc_sh.at[dd.at[slot]], ssem, add=True)
        return 0
    lax.fori_loop(3, _NB, body, 0)

    def drain(j, _):
        pltpu.make_async_copy(ones_r, acc_sh.at[dd.at[0]], ssem).wait()
        return 0
    lax.fori_loop(0, 3, drain, 0)
    plsc.subcore_barrier()

    _acc_writeout(acc_sh, zb, out_hbm.at[0], sid, lo)


@functools.partial(
    pl.kernel,
    out_type=jax.ShapeDtypeStruct((2, _NP, _D), jnp.float32),
    mesh=_mesh,
    scratch_types=[
        pltpu.VMEM((_PKR, 128), jnp.int32),
        pltpu.VMEM((_B,), jnp.int32),
        pltpu.VMEM((_B,), jnp.int32),
        pltpu.VMEM((_B,), jnp.int32),
        pltpu.VMEM((_B,), jnp.int32),
        pltpu.VMEM((_B, _D), jnp.float32),
        pltpu.VMEM((_B, _D), jnp.float32),
        pltpu.VMEM((128, _D), jnp.float32),
        pltpu.VMEM_SHARED((_AR, _D), jnp.float32),
        pltpu.SemaphoreType.DMA,
        pltpu.SemaphoreType.DMA,
    ],
)
def _sn(pk_hbm, tab_hbm, out_hbm, pkv, sa, da, sb, db, ra, rb, zb,
        acc_sh, gsa, gsb):
    cid = lax.axis_index("c")
    sid = lax.axis_index("s")
    lo = cid * _HN

    _zero2d(zb, 128)
    _acc_zero(acc_sh, zb, sid)
    pltpu.sync_copy(pk_hbm.at[sid, pl.ds(0, _PKR)], pkv)
    plsc.subcore_barrier()

    def unpack(j, s_idx, d_idx):
        for k in range(_B // 16):
            flat = j * _B + k * 16
            v = pkv[flat >> 7, pl.ds(flat & 127, 16)]
            s_idx[pl.ds(k * 16, 16)] = v & 0xFFFF
            d = (v >> 16) - lo
            d = jnp.where((d >= 0) & (d < _HN), d, _HN)
            d_idx[pl.ds(k * 16, 16)] = d

    unpack(0, sa, da)
    pltpu.async_copy(tab_hbm.at[sa], ra, gsa)

    def step(j, s_idx, d_idx, rows, gsem, o_s, o_d, o_rows, o_gsem):
        @pl.when(j + 1 < _NB)
        def _():
            unpack(j + 1, o_s, o_d)
            pltpu.async_copy(tab_hbm.at[o_s], o_rows, o_gsem)
        pltpu.make_async_copy(tab_hbm.at[s_idx], rows, gsem).wait()
        pltpu.sync_copy(rows, acc_sh.at[d_idx], add=True)

    def body(j, _):
        @pl.when(lax.rem(j, 2) == 0)
        def _():
            step(j, sa, da, ra, gsa, sb, db, rb, gsb)

        @pl.when(lax.rem(j, 2) == 1)
        def _():
            step(j, sb, db, rb, gsb, sa, da, ra, gsa)
        return 0
    lax.fori_loop(0, _NB, body, 0)
    plsc.subcore_barrier()

    _acc_writeout(acc_sh, zb, out_hbm.at[0], sid, lo)


def _make_seg(lo):
    @functools.partial(
        pl.kernel,
        out_type=jax.ShapeDtypeStruct((2, _NP, _D), jnp.float32),
        mesh=_mesh,
        scratch_types=[
            pltpu.VMEM((_PKR, 128), jnp.int32),
            pltpu.VMEM((_B,), jnp.int32),
            pltpu.VMEM((_B,), jnp.int32),
            pltpu.VMEM((_B,), jnp.int32),
            pltpu.VMEM((_B,), jnp.int32),
            pltpu.VMEM((_B, _D), jnp.float32),
            pltpu.VMEM((_B, _D), jnp.float32),
            pltpu.VMEM((128, _D), jnp.float32),
            pltpu.VMEM_SHARED((_AR, _D), jnp.float32),
            pltpu.SemaphoreType.DMA,
            pltpu.SemaphoreType.DMA,
        ],
    )
    def seg(pk_hbm, tab_hbm, out_hbm, pkv, sa, da, sb, db, ra, rb, zb,
            acc_sh, gsa, gsb):
        cid = lax.axis_index("c")
        sid = lax.axis_index("s")
        soff = cid * _N

        _zero2d(zb, 128)
        _acc_zero(acc_sh, zb, sid)
        pltpu.sync_copy(pk_hbm.at[sid, pl.ds(0, _PKR)], pkv)
        plsc.subcore_barrier()

        def unpack(j, s_idx, d_idx):
            for k in range(_B // 16):
                flat = j * _B + k * 16
                v = pkv[flat >> 7, pl.ds(flat & 127, 16)]
                s_idx[pl.ds(k * 16, 16)] = (v & 0xFFFF) + soff
                d = (v >> 16) - lo
                d = jnp.where((d >= 0) & (d < _HN), d, _HN)
                d_idx[pl.ds(k * 16, 16)] = d

        unpack(0, sa, da)
        pltpu.async_copy(tab_hbm.at[sa], ra, gsa)

        def step(j, s_idx, d_idx, rows, gsem, o_s, o_d, o_rows, o_gsem):
            @pl.when(j + 1 < _NB)
            def _():
                unpack(j + 1, o_s, o_d)
                pltpu.async_copy(tab_hbm.at[o_s], o_rows, o_gsem)
            pltpu.make_async_copy(tab_hbm.at[s_idx], rows, gsem).wait()
            pltpu.sync_copy(rows, acc_sh.at[d_idx], add=True)

        def body(j, _):
            @pl.when(lax.rem(j, 2) == 0)
            def _():
                step(j, sa, da, ra, gsa, sb, db, rb, gsb)

            @pl.when(lax.rem(j, 2) == 1)
            def _():
                step(j, sb, db, rb, gsb, sa, da, ra, gsa)
            return 0
        lax.fori_loop(0, _NB, body, 0)
        plsc.subcore_barrier()

        _acc_writeout(acc_sh, zb, out_hbm.at[cid], sid, lo)

    return seg


_seg_lo = _make_seg(0)
_seg_hi = _make_seg(_HN)


_BLK = 400
_G = _N // _BLK


def _row_spec(w=_D):
    return pl.BlockSpec((_BLK, w), lambda i: (i, 0))


def _full_spec(r, c):
    return pl.BlockSpec((r, c), lambda i: (0, 0))


def _arecip_body(deg_ref, a_ref, inv_ref):
    d = jnp.maximum(deg_ref[:, 0:1], 1.0)
    a_ref[...] = jnp.broadcast_to(lax.rsqrt(d), a_ref.shape)
    inv_ref[...] = jnp.broadcast_to(1.0 / d, inv_ref.shape)


def _arecip(deg16):
    ospec = pl.BlockSpec((_NP // 8, _D), lambda i: (i, 0))
    return pl.pallas_call(
        _arecip_body,
        grid=(8,),
        in_specs=[pl.BlockSpec((_NP // 8, 16), lambda i: (i, 0))],
        out_specs=[ospec, ospec],
        out_shape=[jax.ShapeDtypeStruct((_NP, _D), jnp.float32),
                   jax.ShapeDtypeStruct((_NP, _D), jnp.float32)],
    )(deg16)


def _tca_body(x_ref, w0_ref, b0_ref, a_ref, h_ref, ah_ref):
    h = jnp.dot(x_ref[...], w0_ref[...],
                preferred_element_type=jnp.float32) + b0_ref[...]
    h_ref[...] = h
    ah_ref[...] = a_ref[...] * h


def _tca(x, W0, b0, a_col):
    return pl.pallas_call(
        _tca_body,
        grid=(_G,),
        in_specs=[_row_spec(), _full_spec(_D, _D), _full_spec(1, _D),
                  _row_spec(1)],
        out_specs=[_row_spec(), _row_spec()],
        out_shape=[jax.ShapeDtypeStruct((_N, _D), jnp.float32),
                   jax.ShapeDtypeStruct((_N, _D), jnp.float32)],
    )(x, W0, b0, a_col)


def _elu(v):
    return jnp.where(v > 0, v, jnp.exp(jnp.minimum(v, 0.0)) - 1.0)


def _tclayer_body(h_ref, s0_ref, s1_ref, a_ref, inv_ref, sn_ref,
                  wg_ref, bg_ref, wss_ref, wsn_ref, bs_ref,
                  wgin_ref, bgin_ref, w_ref, ho_ref, aho_ref):
    h = h_ref[...]
    s0 = s0_ref[...]
    a = a_ref[...]
    f32 = jnp.float32
    gcn = (jnp.dot(a * s1_ref[...], wg_ref[...], preferred_element_type=f32)
           + (a * sn_ref[...]) * bg_ref[...])
    sage = (jnp.dot(h, wss_ref[...], preferred_element_type=f32)
            + jnp.dot(inv_ref[...] * s0, wsn_ref[...],
                      preferred_element_type=f32) + bs_ref[...])
    gin = (jnp.dot(h + s0, wgin_ref[...], preferred_element_type=f32)
           + bgin_ref[...])
    xo = (w_ref[0] * _elu(gcn) + w_ref[1] * _elu(sage) + w_ref[2] * _elu(gin))
    ho_ref[...] = xo
    aho_ref[...] = a * xo


def _tclayer(h, S0, S1, a_col, inv_col, sn_col, Wg, bg, Wss, Wsn, bs,
             Wgin, bgin, naw):
    return pl.pallas_call(
        _tclayer_body,
        grid=(_G,),
        in_specs=[_row_spec(), _row_spec(), _row_spec(), _row_spec(1),
                  _row_spec(1), _row_spec(1),
                  _full_spec(_D, _D), _full_spec(1, _D),
                  _full_spec(_D, _D), _full_spec(_D, _D), _full_spec(1, _D),
                  _full_spec(_D, _D), _full_spec(1, _D),
                  pl.BlockSpec(memory_space=pltpu.MemorySpace.SMEM)],
        out_specs=[_row_spec(), _row_spec()],
        out_shape=[jax.ShapeDtypeStruct((_N, _D), jnp.float32),
                   jax.ShapeDtypeStruct((_N, _D), jnp.float32)],
    )(h, S0, S1, a_col, inv_col, sn_col, Wg, bg, Wss, Wsn, bs, Wgin, bgin,
      naw)


def _tcfinal_body(x1_ref, x2_ref, x3_ref, w1_ref, w2_ref, w3_ref, bla_ref,
                  wc_ref, bc_ref, k_ref, out_ref):
    f32 = jnp.float32
    x3 = x3_ref[...]
    sc1 = k_ref[0] * x1_ref[...]
    sc2 = k_ref[1] * x2_ref[...]
    op_max = jnp.maximum(jnp.maximum(x3, sc1), sc2)
    op_cat = (jnp.dot(x3, w1_ref[...], preferred_element_type=f32)
              + jnp.dot(sc1, w2_ref[...], preferred_element_type=f32)
              + jnp.dot(sc2, w3_ref[...], preferred_element_type=f32)
              + bla_ref[...])
    op_mean = (x3 + sc1 + sc2) / 3.0
    x5 = (k_ref[2] * jnp.maximum(op_max, 0.0)
          + k_ref[3] * jnp.maximum(op_cat, 0.0)
          + k_ref[4] * jnp.maximum(op_mean, 0.0))
    out_ref[...] = jnp.dot(x5, wc_ref[...],
                           preferred_element_type=f32) + bc_ref[...]


def _tcfinal(x1, x2, x3, W1, W2, W3, bla, Wc, bc, ks):
    c = Wc.shape[1]
    return pl.pallas_call(
        _tcfinal_body,
        grid=(_G,),
        in_specs=[_row_spec(), _row_spec(), _row_spec(),
                  _full_spec(_D, _D), _full_spec(_D, _D), _full_spec(_D, _D),
                  _full_spec(1, _D), _full_spec(_D, c), _full_spec(1, c),
                  pl.BlockSpec(memory_space=pltpu.MemorySpace.SMEM)],
        out_specs=[pl.BlockSpec((_BLK, c), lambda i: (i, 0))],
        out_shape=[jax.ShapeDtypeStruct((_N, c), jnp.float32)],
    )(x1, x2, x3, W1, W2, W3, bla, Wc, bc, ks)


def kernel(x, edge_index, W0, b0, Wg, bg, Wss, Wsn, bs, Wgin, bgin,
           Wla, bla, Wc, bc, na_alphas, sc_alphas, la_alphas):
    na_w = jax.nn.softmax(na_alphas, axis=-1)
    sc_w = jax.nn.softmax(sc_alphas, axis=-1)
    la_w = jax.nn.softmax(la_alphas, axis=-1)

    src = edge_index[0].astype(jnp.int32)
    dst = edge_index[1].astype(jnp.int32)
    pk = (src + dst * 65536).reshape(16, _EPT)
    pk = jnp.pad(pk, ((0, 0), (0, _PKH * 128 - _EPT)))
    pk = pk.reshape(16, _PKH, 128)

    deg16 = _deg(pk)[0]
    a128, inv128 = _arecip(deg16)
    a_col = a128[:_N, 0:1]
    inv_col = inv128[:_N, 0:1]

    h, ah = _tca(x, W0, b0.reshape(1, _D), a_col)
    sn_tab = jnp.concatenate([a128, a128], axis=0)
    sn_col = _sn(pk, sn_tab)[0][:_N, 0:1]

    xs = []
    for i in range(3):
        tab = jnp.concatenate([h, ah], axis=0)
        Sa = _seg_lo(pk, tab)
        Sb = _seg_hi(pk, tab)
        S0 = jnp.concatenate([Sa[0, :_HN], Sb[0, _HN:_N]], axis=0)
        S1 = jnp.concatenate([Sa[1, :_HN], Sb[1, _HN:_N]], axis=0)
        h, ah = _tclayer(h, S0, S1, a_col, inv_col, sn_col,
                         Wg[i], bg[i].reshape(1, _D), Wss[i], Wsn[i],
                         bs[i].reshape(1, _D), Wgin[i],
                         bgin[i].reshape(1, _D), na_w[i])
        xs.append(h)

    x1, x2, x3 = xs
    ks = jnp.stack([sc_w[0, 1], sc_w[1, 1],
                    la_w[0, 0], la_w[0, 1], la_w[0, 2]])
    (logits,) = _tcfinal(x1, x2, x3, Wla[0:_D], Wla[_D:2 * _D],
                         Wla[2 * _D:3 * _D], bla.reshape(1, _D), Wc,
                         bc.reshape(1, Wc.shape[1]), ks)
    return logits
